# baseline (speedup 1.0000x reference)
import functools

import jax
import jax.numpy as jnp
from jax import lax
from jax.experimental import pallas as pl
from jax.experimental.pallas import tpu as pltpu
from jax.experimental.pallas import tpu_sc as plsc

_LANES = 16


def _make_sc_kernel(B, SEQ):
    info = plsc.get_sparse_core_info()
    NC, NS = info.num_cores, info.num_subcores
    NW = NC * NS
    R = B // NW
    N = R * SEQ
    G = R // _LANES
    assert R % _LANES == 0 and B % NW == 0
    mesh = plsc.VectorSubcoreMesh(core_axis_name="c", subcore_axis_name="s")

    def _build_tidx(idx_v, tidx_v):
        iota = lax.iota(jnp.int32, _LANES)

        def body(j, _):
            for g in range(G):
                rows = (g * _LANES + iota) * SEQ + j
                tidx_v[pl.ds(j * R + g * _LANES, _LANES)] = plsc.load_gather(
                    idx_v, [rows]
                )
            return 0

        lax.fori_loop(0, SEQ, body, 0, unroll=False)

    def _gather(w_hbm, tidx_v, vals_v, sem):
        def issue(j, _):
            pltpu.async_copy(
                w_hbm.at[tidx_v.at[pl.ds(j * R, R)]],
                vals_v.at[pl.ds(j * R, R)],
                sem,
            )
            return 0

        lax.fori_loop(0, SEQ, issue, 0, unroll=False)

    def _drain(w_hbm, tidx_v, vals_v, sem):
        def body(j, _):
            pltpu.make_async_copy(
                w_hbm.at[tidx_v.at[pl.ds(j * R, R)]],
                vals_v.at[pl.ds(j * R, R)],
                sem,
            ).wait()
            return 0

        lax.fori_loop(0, SEQ, body, 0, unroll=False)

    def _reduce(vals_v, out_v):
        def body(j, accs):
            return tuple(
                accs[g] + vals_v[pl.ds(j * R + g * _LANES, _LANES)]
                for g in range(G)
            )

        zeros = jnp.zeros((_LANES,), jnp.float32)
        accs = lax.fori_loop(0, SEQ, body, (zeros,) * G, unroll=False)
        for g in range(G):
            out_v[pl.ds(g * _LANES, _LANES)] = accs[g]

    @functools.partial(
        pl.kernel,
        out_type=jax.ShapeDtypeStruct((B,), jnp.float32),
        mesh=mesh,
        compiler_params=pltpu.CompilerParams(needs_layout_passes=False),
        scratch_types=dict(
            idx_v=pltpu.VMEM((N,), jnp.int32),
            tidx_v=pltpu.VMEM((N,), jnp.int32),
            vals_v=pltpu.VMEM((N,), jnp.float32),
            out_v=pltpu.VMEM((R,), jnp.float32),
            sem=pltpu.SemaphoreType.DMA,
        ),
    )
    def sc_kernel(
        idx_hbm,
        w_hbm,
        out_hbm,
        *,
        idx_v,
        tidx_v,
        vals_v,
        out_v,
        sem,
    ):
        wid = lax.axis_index("s") * NC + lax.axis_index("c")
        base = wid * N
        rbase = wid * R

        pltpu.sync_copy(idx_hbm.at[pl.ds(base, N)], idx_v)
        _build_tidx(idx_v, tidx_v)
        _gather(w_hbm, tidx_v, vals_v, sem)
        _drain(w_hbm, tidx_v, vals_v, sem)
        _reduce(vals_v, out_v)
        pltpu.sync_copy(out_v, out_hbm.at[pl.ds(rbase, R)])

    return sc_kernel


def kernel(inputs_pri, inputs_sec, W_pri, W_sec):
    B, SEQ = inputs_pri.shape
    sc = _make_sc_kernel(B, SEQ)
    out_sec = sc(inputs_sec.reshape(-1).astype(jnp.int32), W_sec[:, 0])
    out_pri = sc(inputs_pri.reshape(-1).astype(jnp.int32), W_pri[:, 0])
    return (out_pri, out_sec)
